# baseline (speedup 1.0000x reference)
import jax
import jax.numpy as jnp
from jax import lax
from jax.experimental import pallas as pl
from jax.experimental.pallas import tpu as pltpu
from jax.experimental.pallas import tpu_sc as plsc

VOCAB_FRESH = 100000
D = 32
B = 16384
L = 50
NF = 16

NC = 2
NS = 16
NW = NC * NS
ROWS_PER_W = B // NW
C = 16
IDS = C * L
NCHUNK = ROWS_PER_W // C
OUT_W = NF + 2 * D
LANES = 16
SCALE = 1.0 / (2 * L)


def _fast_mod(v):
    q = (v.astype(jnp.float32) * jnp.float32(1.0 / VOCAB_FRESH)).astype(
        jnp.int32)
    r = v - q * VOCAB_FRESH
    r = jnp.where(r < 0, r + VOCAB_FRESH, r)
    return jnp.where(r >= VOCAB_FRESH, r - VOCAB_FRESH, r)


def _make_body(apply_mod, accumulate, dw, co_a, co_b):
    nv = dw // LANES

    def body(aux_hbm, aids_hbm, bids_hbm, tbl_hbm, out_hbm,
             ida0, ida1, idb0, idb1,
             rowsa0, rowsa1, rowsb0, rowsb1,
             outv0, outv1,
             sia0, sia1, sib0, sib1,
             sga0, sga1, sgb0, sgb1,
             sax0, sax1):
        ida = (ida0, ida1)
        idb = (idb0, idb1)
        rowsa = (rowsa0, rowsa1)
        rowsb = (rowsb0, rowsb1)
        outv = (outv0, outv1)
        sia = (sia0, sia1)
        sib = (sib0, sib1)
        sga = (sga0, sga1)
        sgb = (sgb0, sgb1)
        sax = (sax0, sax1)

        wid = lax.axis_index("s") * NC + lax.axis_index("c")
        row_base = wid * ROWS_PER_W

        def fetch(g, s):
            row0 = row_base + g * C
            off = row0 * L
            pltpu.async_copy(aids_hbm.at[pl.ds(off, IDS)], ida[s], sia[s])
            pltpu.async_copy(bids_hbm.at[pl.ds(off, IDS)], idb[s], sib[s])
            if accumulate:
                pltpu.async_copy(aux_hbm.at[pl.ds(row0, C)], outv[s], sax[s])
            else:
                pltpu.async_copy(
                    aux_hbm.at[pl.ds(row0, C)],
                    outv[s].at[:, pl.ds(0, NF)], sax[s])

        def mod_gather(g, s):
            pltpu.make_async_copy(
                aids_hbm.at[pl.ds(0, IDS)], ida[s], sia[s]).wait()
            pltpu.make_async_copy(
                bids_hbm.at[pl.ds(0, IDS)], idb[s], sib[s]).wait()
            if apply_mod:
                def mod_body(i, _):
                    off = pl.multiple_of(i * LANES, 8)
                    ida[s][pl.ds(off, LANES)] = _fast_mod(
                        ida[s][pl.ds(off, LANES)])
                    idb[s][pl.ds(off, LANES)] = _fast_mod(
                        idb[s][pl.ds(off, LANES)])
                    return 0
                lax.fori_loop(0, IDS // LANES, mod_body, 0)
            pltpu.async_copy(tbl_hbm.at[ida[s]], rowsa[s], sga[s])
            pltpu.async_copy(tbl_hbm.at[idb[s]], rowsb[s], sgb[s])

        def reduce_rows(rows_ref, base):
            zero = jnp.zeros((LANES,), jnp.float32)

            def rbody(gi, accs):
                accs = list(accs)
                for u in range(5):
                    r = base + gi * 5 + u
                    for k in range(nv):
                        accs[k] = accs[k] + rows_ref[r, pl.ds(k * LANES,
                                                              LANES)]
                return tuple(accs)

            return lax.fori_loop(0, L // 5, rbody, (zero,) * nv)

        def reduce_out(g, s):
            row0 = row_base + g * C
            pltpu.make_async_copy(
                tbl_hbm.at[ida[s]], rowsa[s], sga[s]).wait()
            pltpu.make_async_copy(
                tbl_hbm.at[idb[s]], rowsb[s], sgb[s]).wait()
            if accumulate:
                pltpu.make_async_copy(
                    aux_hbm.at[pl.ds(0, C)], outv[s], sax[s]).wait()
            else:
                pltpu.make_async_copy(
                    aux_hbm.at[pl.ds(0, C)],
                    outv[s].at[:, pl.ds(0, NF)], sax[s]).wait()
            scale = jnp.float32(SCALE)

            def red_body(b, _):
                va = reduce_rows(rowsa[s], b * L)
                vb = reduce_rows(rowsb[s], b * L)
                for k in range(nv):
                    ca = co_a + k * LANES
                    cb = co_b + k * LANES
                    if accumulate:
                        outv[s][b, pl.ds(ca, LANES)] = (
                            outv[s][b, pl.ds(ca, LANES)] + va[k] * scale)
                        outv[s][b, pl.ds(cb, LANES)] = (
                            outv[s][b, pl.ds(cb, LANES)] + vb[k] * scale)
                    else:
                        outv[s][b, pl.ds(ca, LANES)] = va[k] * scale
                        outv[s][b, pl.ds(cb, LANES)] = vb[k] * scale
                return 0

            lax.fori_loop(0, C, red_body, 0)
            pltpu.sync_copy(outv[s], out_hbm.at[pl.ds(row0, C)])

        fetch(0, 0)
        mod_gather(0, 0)
        fetch(1, 1)
        mod_gather(1, 1)

        def pipe(j, _):
            g = j * 2
            reduce_out(g, 0)
            fetch(g + 2, 0)
            mod_gather(g + 2, 0)
            reduce_out(g + 1, 1)
            fetch(g + 3, 1)
            mod_gather(g + 3, 1)
            return 0

        lax.fori_loop(0, NCHUNK // 2 - 1, pipe, 0)
        reduce_out(NCHUNK - 2, 0)
        reduce_out(NCHUNK - 1, 1)

    return body


def _make_kernel(body, dw):
    mesh = plsc.VectorSubcoreMesh(
        core_axis_name="c", subcore_axis_name="s",
        num_cores=NC, num_subcores=NS)
    return pl.kernel(
        body,
        out_type=jax.ShapeDtypeStruct((B, OUT_W), jnp.float32),
        mesh=mesh,
        compiler_params=pltpu.CompilerParams(use_tc_tiling_on_sc=False),
        scratch_types=(
            [pltpu.VMEM((IDS,), jnp.int32)] * 4
            + [pltpu.VMEM((IDS, dw), jnp.float32)] * 4
            + [pltpu.VMEM((C, OUT_W), jnp.float32)] * 2
            + [pltpu.SemaphoreType.DMA] * 10
        ),
    )


@jax.jit
def _run(features, a_flat, b_flat, pre_lo, pre_hi, fresh):
    fresh_fn = _make_kernel(
        _make_body(apply_mod=True, accumulate=False, dw=D,
                   co_a=NF, co_b=NF + D), dw=D)
    pre_lo_fn = _make_kernel(
        _make_body(apply_mod=False, accumulate=True, dw=D // 2,
                   co_a=NF, co_b=NF + D), dw=D // 2)
    pre_hi_fn = _make_kernel(
        _make_body(apply_mod=False, accumulate=True, dw=D // 2,
                   co_a=NF + D // 2, co_b=NF + D + D // 2), dw=D // 2)
    part = fresh_fn(features, a_flat, b_flat, fresh)
    part = pre_lo_fn(part, a_flat, b_flat, pre_lo)
    return pre_hi_fn(part, a_flat, b_flat, pre_hi)


def kernel(features, a_ids, b_ids, pretrained_embeddings, fresh_embeddings):
    a_flat = a_ids.reshape(-1)
    b_flat = b_ids.reshape(-1)
    pre_lo = pretrained_embeddings[:, : D // 2]
    pre_hi = pretrained_embeddings[:, D // 2:]
    return _run(features, a_flat, b_flat, pre_lo, pre_hi, fresh_embeddings)


# speedup vs baseline: 1.1137x; 1.1137x over previous
import jax
import jax.numpy as jnp
import numpy as np
from jax import lax
from jax.experimental import pallas as pl
from jax.experimental.pallas import tpu as pltpu
from jax.experimental.pallas import tpu_sc as plsc

VOCAB_FRESH = 100000
D = 32
B = 16384
L = 50
NF = 16

NC = 2
NS = 16
NW = NC * NS
ROWS_PER_W = B // NW
C = 16
IDS = C * L
NCHUNK = ROWS_PER_W // C
OUT_W = NF + 2 * D
LANES = 16
SCALE = 1.0 / (2 * L)
HALF = D // 2


def _fast_mod(v):
    q = (v.astype(jnp.float32) * jnp.float32(1.0 / VOCAB_FRESH)).astype(
        jnp.int32)
    r = v - q * VOCAB_FRESH
    r = jnp.where(r < 0, r + VOCAB_FRESH, r)
    return jnp.where(r >= VOCAB_FRESH, r - VOCAB_FRESH, r)


def _make_body(apply_mod, accumulate):
    def body(aux_hbm, aids_hbm, bids_hbm, tbl_hbm, out_hbm,
             ida0, ida1, idb0, idb1,
             rowsa0, rowsa1, rowsb0, rowsb1,
             outv0, outv1,
             sia0, sia1, sib0, sib1,
             sga0, sga1, sgb0, sgb1,
             sax0, sax1):
        ida = (ida0, ida1)
        idb = (idb0, idb1)
        rowsa = (rowsa0, rowsa1)
        rowsb = (rowsb0, rowsb1)
        outv = (outv0, outv1)
        sia = (sia0, sia1)
        sib = (sib0, sib1)
        sga = (sga0, sga1)
        sgb = (sgb0, sgb1)
        sax = (sax0, sax1)

        wid = lax.axis_index("s") * NC + lax.axis_index("c")
        row_base = wid * ROWS_PER_W

        def fetch(g, s):
            row0 = row_base + g * C
            off = row0 * L
            pltpu.async_copy(aids_hbm.at[pl.ds(off, IDS)], ida[s], sia[s])
            pltpu.async_copy(bids_hbm.at[pl.ds(off, IDS)], idb[s], sib[s])
            if accumulate:
                pltpu.async_copy(aux_hbm.at[pl.ds(row0, C)], outv[s], sax[s])
            else:
                pltpu.async_copy(
                    aux_hbm.at[pl.ds(row0, C)],
                    outv[s].at[:, pl.ds(0, NF)], sax[s])

        def mod_gather(g, s):
            pltpu.make_async_copy(
                aids_hbm.at[pl.ds(0, IDS)], ida[s], sia[s]).wait()
            pltpu.make_async_copy(
                bids_hbm.at[pl.ds(0, IDS)], idb[s], sib[s]).wait()
            if apply_mod:
                def mod_body(i, _):
                    off = pl.multiple_of(i * LANES, 8)
                    ida[s][pl.ds(off, LANES)] = _fast_mod(
                        ida[s][pl.ds(off, LANES)])
                    idb[s][pl.ds(off, LANES)] = _fast_mod(
                        idb[s][pl.ds(off, LANES)])
                    return 0
                lax.fori_loop(0, IDS // LANES, mod_body, 0)
            pltpu.async_copy(tbl_hbm.at[ida[s]], rowsa[s], sga[s])
            pltpu.async_copy(tbl_hbm.at[idb[s]], rowsb[s], sgb[s])

        def reduce_rows(rows_ref, base):
            zero = jnp.zeros((LANES,), jnp.float32)
            mask_hi = jnp.full((LANES,), -65536, jnp.int32)

            def rbody(gi, accs):
                a0, a1 = accs
                for u in range(5):
                    r = base + gi * 5 + u
                    w = rows_ref[r, pl.ds(0, LANES)]
                    lo = lax.bitcast_convert_type(
                        lax.shift_left(w, 16), jnp.float32)
                    hi = lax.bitcast_convert_type(
                        lax.bitwise_and(w, mask_hi), jnp.float32)
                    a0 = a0 + lo
                    a1 = a1 + hi
                return (a0, a1)

            return lax.fori_loop(0, L // 5, rbody, (zero, zero))

        def reduce_out(g, s):
            row0 = row_base + g * C
            pltpu.make_async_copy(
                tbl_hbm.at[ida[s]], rowsa[s], sga[s]).wait()
            pltpu.make_async_copy(
                tbl_hbm.at[idb[s]], rowsb[s], sgb[s]).wait()
            if accumulate:
                pltpu.make_async_copy(
                    aux_hbm.at[pl.ds(0, C)], outv[s], sax[s]).wait()
            else:
                pltpu.make_async_copy(
                    aux_hbm.at[pl.ds(0, C)],
                    outv[s].at[:, pl.ds(0, NF)], sax[s]).wait()
            scale = jnp.float32(SCALE)

            def red_body(b, _):
                a0, a1 = reduce_rows(rowsa[s], b * L)
                b0, b1 = reduce_rows(rowsb[s], b * L)
                if accumulate:
                    outv[s][b, pl.ds(NF, LANES)] = (
                        outv[s][b, pl.ds(NF, LANES)] + a0 * scale)
                    outv[s][b, pl.ds(NF + HALF, LANES)] = (
                        outv[s][b, pl.ds(NF + HALF, LANES)] + a1 * scale)
                    outv[s][b, pl.ds(NF + D, LANES)] = (
                        outv[s][b, pl.ds(NF + D, LANES)] + b0 * scale)
                    outv[s][b, pl.ds(NF + D + HALF, LANES)] = (
                        outv[s][b, pl.ds(NF + D + HALF, LANES)] + b1 * scale)
                else:
                    outv[s][b, pl.ds(NF, LANES)] = a0 * scale
                    outv[s][b, pl.ds(NF + HALF, LANES)] = a1 * scale
                    outv[s][b, pl.ds(NF + D, LANES)] = b0 * scale
                    outv[s][b, pl.ds(NF + D + HALF, LANES)] = b1 * scale
                return 0

            lax.fori_loop(0, C, red_body, 0)
            pltpu.sync_copy(outv[s], out_hbm.at[pl.ds(row0, C)])

        fetch(0, 0)
        mod_gather(0, 0)
        fetch(1, 1)
        mod_gather(1, 1)

        def pipe(j, _):
            g = j * 2
            reduce_out(g, 0)
            fetch(g + 2, 0)
            mod_gather(g + 2, 0)
            reduce_out(g + 1, 1)
            fetch(g + 3, 1)
            mod_gather(g + 3, 1)
            return 0

        lax.fori_loop(0, NCHUNK // 2 - 1, pipe, 0)
        reduce_out(NCHUNK - 2, 0)
        reduce_out(NCHUNK - 1, 1)

    return body


def _make_kernel(body, row_words, row_dtype):
    mesh = plsc.VectorSubcoreMesh(
        core_axis_name="c", subcore_axis_name="s",
        num_cores=NC, num_subcores=NS)
    return pl.kernel(
        body,
        out_type=jax.ShapeDtypeStruct((B, OUT_W), jnp.float32),
        mesh=mesh,
        compiler_params=pltpu.CompilerParams(use_tc_tiling_on_sc=False),
        scratch_types=(
            [pltpu.VMEM((IDS,), jnp.int32)] * 4
            + [pltpu.VMEM((IDS, row_words), row_dtype)] * 4
            + [pltpu.VMEM((C, OUT_W), jnp.float32)] * 2
            + [pltpu.SemaphoreType.DMA] * 10
        ),
    )


_UNPERM = np.arange(OUT_W)
for _d in range(D):
    _col = (NF + _d // 2) if _d % 2 == 0 else (NF + HALF + _d // 2)
    _UNPERM[NF + _d] = _col
    _UNPERM[NF + D + _d] = _col + D


@jax.jit
def _run(features, a_flat, b_flat, pre_w, fresh_w):
    fresh_fn = _make_kernel(
        _make_body(apply_mod=True, accumulate=False), HALF, jnp.int32)
    pre_fn = _make_kernel(
        _make_body(apply_mod=False, accumulate=True), HALF, jnp.int32)
    part = fresh_fn(features, a_flat, b_flat, fresh_w)
    out = pre_fn(part, a_flat, b_flat, pre_w)
    return out[:, _UNPERM]


def kernel(features, a_ids, b_ids, pretrained_embeddings, fresh_embeddings):
    a_flat = a_ids.reshape(-1)
    b_flat = b_ids.reshape(-1)
    pre_w = lax.bitcast_convert_type(
        pretrained_embeddings.astype(jnp.bfloat16).reshape(-1, HALF, 2),
        jnp.int32)
    fresh_w = lax.bitcast_convert_type(
        fresh_embeddings.astype(jnp.bfloat16).reshape(-1, HALF, 2),
        jnp.int32)
    return _run(features, a_flat, b_flat, pre_w, fresh_w)
